# baseline (speedup 1.0000x reference)
import functools

import jax
import jax.numpy as jnp
from jax import lax
from jax.experimental import pallas as pl
from jax.experimental.pallas import tpu as pltpu
from jax.experimental.pallas import tpu_sc as plsc

B = 4
N = 8192
M = 8192
NW = 32
QPW = (B * N) // NW
WPB = N // QPW
L = 16


def _nn_body(q_hbm, r_hbm, out_hbm, qx, qy, qz, rx, ry, rz, oi):
    c = lax.axis_index("c")
    s = lax.axis_index("s")
    wid = s * 2 + c
    b = wid // WPB
    qbase = (wid % WPB) * QPW

    pltpu.sync_copy(q_hbm.at[pl.ds(b * 3 * N + 0 * N + qbase, QPW)], qx)
    pltpu.sync_copy(q_hbm.at[pl.ds(b * 3 * N + 1 * N + qbase, QPW)], qy)
    pltpu.sync_copy(q_hbm.at[pl.ds(b * 3 * N + 2 * N + qbase, QPW)], qz)
    pltpu.sync_copy(r_hbm.at[pl.ds(b * 3 * M + 0 * M, M)], rx)
    pltpu.sync_copy(r_hbm.at[pl.ds(b * 3 * M + 1 * M, M)], ry)
    pltpu.sync_copy(r_hbm.at[pl.ds(b * 3 * M + 2 * M, M)], rz)

    def per_group(g, carry):
        qxv = qx[pl.ds(g * L, L)]
        qyv = qy[pl.ds(g * L, L)]
        qzv = qz[pl.ds(g * L, L)]

        def per_refvec(j, mc):
            m, mi = mc
            rxv = rx[pl.ds(j * L, L)]
            ryv = ry[pl.ds(j * L, L)]
            rzv = rz[pl.ds(j * L, L)]
            base = jnp.full((L,), j * L, jnp.int32)
            for t in range(L):
                tv = jnp.full((L,), t, jnp.int32)
                dx = qxv - jnp.take(rxv, tv)
                dy = qyv - jnp.take(ryv, tv)
                dz = qzv - jnp.take(rzv, tv)
                d = dx * dx + dy * dy + dz * dz
                p = d < m
                m = jnp.minimum(m, d)
                mi = jnp.where(p, base + t, mi)
            return m, mi

        m0 = jnp.full((L,), jnp.inf, jnp.float32)
        mi0 = jnp.zeros((L,), jnp.int32)
        _, mi = lax.fori_loop(0, M // L, per_refvec, (m0, mi0))
        oi[pl.ds(g * L, L)] = mi
        return carry

    lax.fori_loop(0, QPW // L, per_group, 0)
    pltpu.sync_copy(oi, out_hbm.at[pl.ds(b * N + qbase, QPW)])


_sc_nn = functools.partial(
    pl.kernel,
    out_type=jax.ShapeDtypeStruct((B * N,), jnp.int32),
    mesh=plsc.VectorSubcoreMesh(core_axis_name="c", subcore_axis_name="s"),
    scratch_types=[
        pltpu.VMEM((QPW,), jnp.float32),
        pltpu.VMEM((QPW,), jnp.float32),
        pltpu.VMEM((QPW,), jnp.float32),
        pltpu.VMEM((M,), jnp.float32),
        pltpu.VMEM((M,), jnp.float32),
        pltpu.VMEM((M,), jnp.float32),
        pltpu.VMEM((QPW,), jnp.int32),
    ],
)(_nn_body)


def kernel(S1, S2):
    q = S1.transpose(0, 2, 1).reshape(-1)
    r = S2.transpose(0, 2, 1).reshape(-1)
    idx = _sc_nn(q, r)
    return idx.reshape(B, N).astype(jnp.int64)


# speedup vs baseline: 2.0939x; 2.0939x over previous
import functools

import jax
import jax.numpy as jnp
from jax import lax
from jax.experimental import pallas as pl
from jax.experimental.pallas import tpu as pltpu
from jax.experimental.pallas import tpu_sc as plsc

B = 4
N = 8192
M = 8192
NW = 32
QPW = (B * N) // NW
WPB = N // QPW
L = 16


def _sc_body(q_hbm, r_hbm, out_hbm, qx, qy, qz, rx, ry, rz, oi):
    c = lax.axis_index("c")
    s = lax.axis_index("s")
    wid = s * 2 + c
    b = wid // WPB
    qbase = (wid % WPB) * QPW

    pltpu.sync_copy(q_hbm.at[pl.ds(b * 3 * N + 0 * N + qbase, QPW)], qx)
    pltpu.sync_copy(q_hbm.at[pl.ds(b * 3 * N + 1 * N + qbase, QPW)], qy)
    pltpu.sync_copy(q_hbm.at[pl.ds(b * 3 * N + 2 * N + qbase, QPW)], qz)
    pltpu.sync_copy(r_hbm.at[pl.ds(b * 3 * M + 0 * M, M)], rx)
    pltpu.sync_copy(r_hbm.at[pl.ds(b * 3 * M + 1 * M, M)], ry)
    pltpu.sync_copy(r_hbm.at[pl.ds(b * 3 * M + 2 * M, M)], rz)

    def per_group(g, carry):
        qxv = qx[pl.ds(g * L, L)]
        qyv = qy[pl.ds(g * L, L)]
        qzv = qz[pl.ds(g * L, L)]

        def per_refvec(j, mc):
            m, mi = mc
            rxv = rx[pl.ds(j * L, L)]
            ryv = ry[pl.ds(j * L, L)]
            rzv = rz[pl.ds(j * L, L)]
            base = jnp.full((L,), j * L, jnp.int32)
            for t in range(L):
                tv = jnp.full((L,), t, jnp.int32)
                dx = qxv - jnp.take(rxv, tv)
                dy = qyv - jnp.take(ryv, tv)
                dz = qzv - jnp.take(rzv, tv)
                d = dx * dx + dy * dy + dz * dz
                p = d < m
                m = jnp.minimum(m, d)
                mi = jnp.where(p, base + t, mi)
            return m, mi

        m0 = jnp.full((L,), jnp.inf, jnp.float32)
        mi0 = jnp.zeros((L,), jnp.int32)
        _, mi = lax.fori_loop(0, M // L, per_refvec, (m0, mi0))
        oi[pl.ds(g * L, L)] = mi
        return carry

    lax.fori_loop(0, QPW // L, per_group, 0)
    pltpu.sync_copy(oi, out_hbm.at[pl.ds(b * N + qbase, QPW)])


_sc_nn = functools.partial(
    pl.kernel,
    out_type=jax.ShapeDtypeStruct((B * N,), jnp.int32),
    mesh=plsc.VectorSubcoreMesh(core_axis_name="c", subcore_axis_name="s"),
    scratch_types=[
        pltpu.VMEM((QPW,), jnp.float32),
        pltpu.VMEM((QPW,), jnp.float32),
        pltpu.VMEM((QPW,), jnp.float32),
        pltpu.VMEM((M,), jnp.float32),
        pltpu.VMEM((M,), jnp.float32),
        pltpu.VMEM((M,), jnp.float32),
        pltpu.VMEM((QPW,), jnp.int32),
    ],
)(_sc_body)


QT = 1024
CH = 1024


def _tc_body(qx_ref, qy_ref, qz_ref, rx_ref, ry_ref, rz_ref, out_ref, m_ref, mi_ref):
    r = pl.program_id(2)

    @pl.when(r == 0)
    def _():
        m_ref[...] = jnp.full((8, 128), jnp.inf, jnp.float32)
        mi_ref[...] = jnp.zeros((8, 128), jnp.int32)

    qxt = qx_ref[0, 0]
    qyt = qy_ref[0, 0]
    qzt = qz_ref[0, 0]
    base = r * CH

    def step(j, mc):
        m, mi = mc
        dx = qxt - rx_ref[0, 0, j]
        dy = qyt - ry_ref[0, 0, j]
        dz = qzt - rz_ref[0, 0, j]
        d = dx * dx + dy * dy + dz * dz
        p = d < m
        m = jnp.minimum(m, d)
        mi = jnp.where(p, jnp.full((8, 128), base + j, jnp.int32), mi)
        return m, mi

    m, mi = lax.fori_loop(0, CH, step, (m_ref[...], mi_ref[...]), unroll=8)
    m_ref[...] = m
    mi_ref[...] = mi

    @pl.when(r == M // CH - 1)
    def _():
        out_ref[0] = mi


def _tc_nn(q, r, nb):
    qt = q.reshape(nb, 3, N // 128, 128)
    nrc = M // CH
    rt = r.reshape(nb * 3 * nrc, 1, CH)
    grid = (nb, N // QT, nrc)

    def rmap(c):
        return lambda b, i, j: ((b * 3 + c) * nrc + j, 0, 0)

    out = pl.pallas_call(
        _tc_body,
        grid=grid,
        in_specs=[
            pl.BlockSpec((1, 1, QT // 128, 128), lambda b, i, j: (b, 0, i, 0)),
            pl.BlockSpec((1, 1, QT // 128, 128), lambda b, i, j: (b, 1, i, 0)),
            pl.BlockSpec((1, 1, QT // 128, 128), lambda b, i, j: (b, 2, i, 0)),
            pl.BlockSpec((1, 1, CH), rmap(0), memory_space=pltpu.SMEM),
            pl.BlockSpec((1, 1, CH), rmap(1), memory_space=pltpu.SMEM),
            pl.BlockSpec((1, 1, CH), rmap(2), memory_space=pltpu.SMEM),
        ],
        out_specs=pl.BlockSpec((1, QT // 128, 128), lambda b, i, j: (b, i, 0)),
        out_shape=jax.ShapeDtypeStruct((nb, N // 128, 128), jnp.int32),
        scratch_shapes=[
            pltpu.VMEM((8, 128), jnp.float32),
            pltpu.VMEM((8, 128), jnp.int32),
        ],
        compiler_params=pltpu.CompilerParams(
            dimension_semantics=("arbitrary", "arbitrary", "arbitrary"),
        ),
    )(qt, qt, qt, rt, rt, rt)
    return out.reshape(nb, N)


def kernel(S1, S2):
    q = S1.transpose(0, 2, 1)
    r = S2.transpose(0, 2, 1)
    idx = _tc_nn(q, r, B)
    return idx.astype(jnp.int64)
